# baseline (speedup 1.0000x reference)
import jax
import jax.numpy as jnp
from jax import lax
from jax.experimental import pallas as pl
from jax.experimental.pallas import tpu as pltpu
from jax.experimental.pallas import tpu_sc as plsc

_N = 10000
_D = 256
_E = 160000
_H = _D // 2
_NCORE = 2
_NSUB = 16
_CH = 80
_EPS = _E // _NSUB
_CPB = 25
_NBLK = _EPS // (_CPB * _CH)
_NPAD = 10240
_NPS = _NPAD // _NSUB
_BN = 1000

_f32 = jnp.float32

_MESH = plsc.VectorSubcoreMesh(core_axis_name="c", subcore_axis_name="s")


def _sc_agg_body(x2, src4, dst4, out, src_v, dst_v, rows_v, acc_sh, sem):
  core = lax.axis_index("c")
  sid = lax.axis_index("s")
  row0 = sid * _NPS

  @pl.loop(0, _CH)
  def _(r):
    @pl.loop(0, _H, step=16)
    def _(c):
      rows_v[r, pl.ds(c, 16)] = jnp.zeros((16,), _f32)

  @pl.loop(0, _NPS, step=_CH)
  def _(r):
    pltpu.sync_copy(rows_v, acc_sh.at[pl.ds(row0 + r, _CH)])

  plsc.subcore_barrier()

  @pl.loop(0, _NBLK)
  def _(k):
    pltpu.sync_copy(src4.at[sid, k], src_v)
    pltpu.sync_copy(dst4.at[sid, k], dst_v)

    @pl.loop(0, _CPB)
    def _(j):
      @pl.loop(0, _CH, step=16)
      def _(c):
        v = src_v[j, pl.ds(c, 16)]
        src_v[j, pl.ds(c, 16)] = v * 2 + core

    @pl.loop(0, _CPB)
    def _(j):
      pltpu.sync_copy(x2.at[src_v.at[j]], rows_v)
      pltpu.sync_copy(rows_v, acc_sh.at[dst_v.at[j]], add=True)

  plsc.subcore_barrier()

  pltpu.sync_copy(acc_sh.at[pl.ds(row0, _NPS)],
                  out.at[core, pl.ds(row0, _NPS)])


_sc_agg = pl.kernel(
    _sc_agg_body,
    out_type=jax.ShapeDtypeStruct((_NCORE, _NPAD, _H), _f32),
    mesh=_MESH,
    scratch_types=[
        pltpu.VMEM((_CPB, _CH), jnp.int32),
        pltpu.VMEM((_CPB, _CH), jnp.int32),
        pltpu.VMEM((_CH, _H), _f32),
        pltpu.VMEM_SHARED((_NPAD, _H), _f32),
        pltpu.SemaphoreType.DMA,
    ],
)


def _tc_pre(x, Wr, bl2):
  def tc_body(x_ref, wr_ref, bl_ref, o_ref):
    o_ref[...] = (jnp.dot(x_ref[...], wr_ref[...], preferred_element_type=_f32,
                          precision=lax.Precision.HIGHEST) + bl_ref[...])

  return pl.pallas_call(
      tc_body,
      grid=(_N // _BN,),
      in_specs=[pl.BlockSpec((_BN, _D), lambda i: (i, 0)),
                pl.BlockSpec((_D, _D), lambda i: (0, 0)),
                pl.BlockSpec((1, _D), lambda i: (0, 0))],
      out_specs=pl.BlockSpec((_BN, _D), lambda i: (i, 0)),
      out_shape=jax.ShapeDtypeStruct((_N, _D), _f32),
  )(x, Wr, bl2)


def _tc_post(agg2, cnt, xr, Wl):
  def tc_body(a_ref, cnt_ref, xr_ref, wl_ref, o_ref):
    recip = 1.0 / jnp.maximum(cnt_ref[:, 0:1], 1.0)
    a0 = a_ref[0] * recip
    a1 = a_ref[1] * recip
    y = jnp.dot(a0, wl_ref[0:_H, :], preferred_element_type=_f32,
                precision=lax.Precision.HIGHEST)
    y = y + jnp.dot(a1, wl_ref[_H:_D, :], preferred_element_type=_f32,
                    precision=lax.Precision.HIGHEST)
    y = y + xr_ref[...]
    m = jnp.max(y, axis=-1, keepdims=True)
    e = jnp.exp(y - m)
    o_ref[...] = e / jnp.sum(e, axis=-1, keepdims=True)

  return pl.pallas_call(
      tc_body,
      grid=(_N // _BN,),
      in_specs=[pl.BlockSpec((_NCORE, _BN, _H), lambda i: (0, i, 0)),
                pl.BlockSpec((_BN, 16), lambda i: (i, 0)),
                pl.BlockSpec((_BN, _D), lambda i: (i, 0)),
                pl.BlockSpec((_D, _D), lambda i: (0, 0))],
      out_specs=pl.BlockSpec((_BN, _D), lambda i: (i, 0)),
      out_shape=jax.ShapeDtypeStruct((_N, _D), _f32),
  )(agg2, cnt, xr, Wl)


def kernel(x, edge_index, Wl0, bl0, Wr0, Wl1, bl1, Wr1, Wl2, bl2, Wr2):
  src4 = edge_index[0].reshape(_NSUB, _NBLK, _CPB, _CH)
  dst4 = edge_index[1].reshape(_NSUB, _NBLK, _CPB, _CH)
  params = [(Wl0, bl0, Wr0), (Wl1, bl1, Wr1), (Wl2, bl2, Wr2)]
  ones_x2 = jnp.ones((_NCORE * _N, _H), _f32)
  src4_zero = jnp.zeros(src4.shape, jnp.int32)
  cnt = _sc_agg(ones_x2, src4_zero, dst4)[0, :, :16]
  h = x
  for Wl, bl, Wr in params:
    x2 = h.reshape(_NCORE * _N, _H)
    agg2 = _sc_agg(x2, src4, dst4)
    xr = _tc_pre(h, Wr, bl.reshape(1, _D))
    h = _tc_post(agg2, cnt, xr, Wl)
  return h


# speedup vs baseline: 7.0554x; 7.0554x over previous
import jax
import jax.numpy as jnp
from jax import lax
from jax.experimental import pallas as pl
from jax.experimental.pallas import tpu as pltpu
from jax.experimental.pallas import tpu_sc as plsc

_N = 10000
_D = 256
_E = 160000
_H = _D // 2
_NCORE = 2
_NSUB = 16
_CH = 80
_EPS = _E // _NSUB
_CPB = 25
_NBLK = _EPS // (_CPB * _CH)
_NPAD = 10240
_NPS = _NPAD // _NSUB
_BN = 1000

_f32 = jnp.float32

_MESH = plsc.VectorSubcoreMesh(core_axis_name="c", subcore_axis_name="s")


def _sc_agg_body(x2, src4, dst4, out, src_v, dst_v, rows_v, acc_sh, sem):
  core = lax.axis_index("c")
  sid = lax.axis_index("s")
  row0 = sid * _NPS

  @pl.loop(0, _CH)
  def _(r):
    @pl.loop(0, _H, step=16)
    def _(c):
      rows_v[r, pl.ds(c, 16)] = jnp.zeros((16,), _f32)

  @pl.loop(0, _NPS, step=_CH)
  def _(r):
    pltpu.sync_copy(rows_v, acc_sh.at[pl.ds(row0 + r, _CH)])

  plsc.subcore_barrier()

  @pl.loop(0, _NBLK)
  def _(k):
    pltpu.sync_copy(src4.at[sid, k], src_v)
    pltpu.sync_copy(dst4.at[sid, k], dst_v)

    @pl.loop(0, _CPB)
    def _(j):
      @pl.loop(0, _CH, step=16)
      def _(c):
        v = src_v[j, pl.ds(c, 16)]
        src_v[j, pl.ds(c, 16)] = v * 2 + core

    @pl.loop(0, _CPB)
    def _(j):
      pltpu.sync_copy(x2.at[src_v.at[j]], rows_v)
      pltpu.sync_copy(rows_v, acc_sh.at[dst_v.at[j]], add=True)

  plsc.subcore_barrier()

  pltpu.sync_copy(acc_sh.at[pl.ds(row0, _NPS)],
                  out.at[core, pl.ds(row0, _NPS)])


_sc_agg = pl.kernel(
    _sc_agg_body,
    out_type=jax.ShapeDtypeStruct((_NCORE, _NPAD, _H), _f32),
    mesh=_MESH,
    scratch_types=[
        pltpu.VMEM((_CPB, _CH), jnp.int32),
        pltpu.VMEM((_CPB, _CH), jnp.int32),
        pltpu.VMEM((_CH, _H), _f32),
        pltpu.VMEM_SHARED((_NPAD, _H), _f32),
        pltpu.SemaphoreType.DMA,
    ],
)


def _tc_pre(x, Wr, bl2):
  def tc_body(x_ref, wr_ref, bl_ref, o_ref):
    o_ref[...] = (jnp.dot(x_ref[...], wr_ref[...], preferred_element_type=_f32,
                          precision=lax.Precision.HIGHEST) + bl_ref[...])

  return pl.pallas_call(
      tc_body,
      grid=(_N // _BN,),
      in_specs=[pl.BlockSpec((_BN, _D), lambda i: (i, 0)),
                pl.BlockSpec((_D, _D), lambda i: (0, 0)),
                pl.BlockSpec((1, _D), lambda i: (0, 0))],
      out_specs=pl.BlockSpec((_BN, _D), lambda i: (i, 0)),
      out_shape=jax.ShapeDtypeStruct((_N, _D), _f32),
  )(x, Wr, bl2)


def _tc_post(agg2, cnt, xr, Wl):
  def tc_body(a_ref, cnt_ref, xr_ref, wl_ref, o_ref):
    recip = 1.0 / jnp.maximum(cnt_ref[:, 0:1], 1.0)
    a0 = a_ref[0] * recip
    a1 = a_ref[1] * recip
    y = jnp.dot(a0, wl_ref[0:_H, :], preferred_element_type=_f32,
                precision=lax.Precision.HIGHEST)
    y = y + jnp.dot(a1, wl_ref[_H:_D, :], preferred_element_type=_f32,
                    precision=lax.Precision.HIGHEST)
    y = y + xr_ref[...]
    m = jnp.max(y, axis=-1, keepdims=True)
    e = jnp.exp(y - m)
    o_ref[...] = e / jnp.sum(e, axis=-1, keepdims=True)

  return pl.pallas_call(
      tc_body,
      grid=(_N // _BN,),
      in_specs=[pl.BlockSpec((_NCORE, _BN, _H), lambda i: (0, i, 0)),
                pl.BlockSpec((_BN, 16), lambda i: (i, 0)),
                pl.BlockSpec((_BN, _D), lambda i: (i, 0)),
                pl.BlockSpec((_D, _D), lambda i: (0, 0))],
      out_specs=pl.BlockSpec((_BN, _D), lambda i: (i, 0)),
      out_shape=jax.ShapeDtypeStruct((_N, _D), _f32),
  )(agg2, cnt, xr, Wl)


def kernel(x, edge_index, Wl0, bl0, Wr0, Wl1, bl1, Wr1, Wl2, bl2, Wr2):
  src4 = edge_index[0].reshape(_NSUB, _NBLK, _CPB, _CH)
  dst4 = edge_index[1].reshape(_NSUB, _NBLK, _CPB, _CH)
  params = [(Wl0, bl0, Wr0), (Wl1, bl1, Wr1), (Wl2, bl2, Wr2)]
  ones_x2 = jnp.ones((_NCORE * _N, _H), _f32)
  cnt = _sc_agg(ones_x2, src4, dst4)[0, :, :16]
  h = x
  for Wl, bl, Wr in params:
    x2 = h.reshape(_NCORE * _N, _H)
    agg2 = _sc_agg(x2, src4, dst4)
    xr = _tc_pre(h, Wr, bl.reshape(1, _D))
    h = _tc_post(agg2, cnt, xr, Wl)
  return h
